# baseline (speedup 1.0000x reference)
import functools

import jax
import jax.numpy as jnp
from jax import lax
from jax.experimental import pallas as pl
from jax.experimental.pallas import tpu as pltpu
from jax.experimental.pallas import tpu_sc as plsc

_L = 16
_NW = 32
_H_TC = 24


def _sc_loss(en_hbm, de_hbm, out_hbm, enb, deb, stage, sem0, sem1):
    B, H, W, C = en_hbm.shape
    wid = lax.axis_index("s") * 2 + lax.axis_index("c")
    b = wid // 2
    h_sc = H - _H_TC
    h0 = _H_TC + (wid % 2) * (h_sc // 2)
    nchunk = h_sc // 2
    nvec = C // _L

    def start(g, par, sem):
        pltpu.make_async_copy(en_hbm.at[b, h0 + g], enb.at[par], sem).start()
        pltpu.make_async_copy(de_hbm.at[b, h0 + g], deb.at[par], sem).start()

    def wait(par, sem):
        pltpu.make_async_copy(en_hbm.at[b, h0], enb.at[par], sem).wait()
        pltpu.make_async_copy(de_hbm.at[b, h0], deb.at[par], sem).wait()

    def compute(par, accs):
        def row(i, accs):
            d, e, f = accs
            for j in range(nvec):
                ev = enb[par, i, pl.ds(j * _L, _L)]
                dv = deb[par, i, pl.ds(j * _L, _L)]
                d = d + ev * dv
                e = e + ev * ev
                f = f + dv * dv
            return (d, e, f)

        return lax.fori_loop(0, W, row, accs)

    start(0, 0, sem0)
    start(1, 1, sem1)
    zero = jnp.zeros((_L,), jnp.float32)
    accs = (zero, zero, zero)

    def outer(g2, accs):
        wait(0, sem0)
        accs = compute(0, accs)

        @pl.when(g2 < nchunk // 2 - 1)
        def _():
            start(2 * g2 + 2, 0, sem0)

        wait(1, sem1)
        accs = compute(1, accs)

        @pl.when(g2 < nchunk // 2 - 1)
        def _():
            start(2 * g2 + 3, 1, sem1)

        return accs

    d, e, f = lax.fori_loop(0, nchunk // 2, outer, accs)
    stage[0, :] = d
    stage[1, :] = e
    stage[2, :] = f
    pltpu.sync_copy(stage, out_hbm.at[wid])


def _tc_loss(en_ref, de_ref, out_ref):
    en = en_ref[0]
    de = de_ref[0]
    out_ref[0, 0, 0] = jnp.sum(en * de)
    out_ref[0, 0, 1] = jnp.sum(en * en)
    out_ref[0, 0, 2] = jnp.sum(de * de)


def kernel(encoder_features, decoder_features, global_step):
    B, C, H, W = encoder_features.shape
    en = jnp.transpose(encoder_features, (0, 2, 3, 1))
    de = jnp.transpose(decoder_features, (0, 2, 3, 1))

    mesh = plsc.VectorSubcoreMesh(core_axis_name="c", subcore_axis_name="s")
    sc_call = functools.partial(
        pl.kernel,
        mesh=mesh,
        out_type=jax.ShapeDtypeStruct((_NW, 3, _L), jnp.float32),
        scratch_types=[
            pltpu.VMEM((2, W, C), jnp.float32),
            pltpu.VMEM((2, W, C), jnp.float32),
            pltpu.VMEM((3, _L), jnp.float32),
            pltpu.SemaphoreType.DMA,
            pltpu.SemaphoreType.DMA,
        ],
    )(_sc_loss)
    sc_part = sc_call(en, de)

    tc_part = pl.pallas_call(
        _tc_loss,
        grid=(B,),
        in_specs=[
            pl.BlockSpec((1, _H_TC, W, C), lambda b: (b, 0, 0, 0)),
            pl.BlockSpec((1, _H_TC, W, C), lambda b: (b, 0, 0, 0)),
        ],
        out_specs=pl.BlockSpec(
            (1, 1, 3), lambda b: (b, 0, 0), memory_space=pltpu.SMEM
        ),
        out_shape=jax.ShapeDtypeStruct((B, 1, 3), jnp.float32),
    )(en, de)[:, 0, :]

    sc_per_batch = sc_part.sum(-1).reshape(B, 2, 3).sum(1)
    tot = sc_per_batch + tc_part
    dot, na2, nb2 = tot[:, 0], tot[:, 1], tot[:, 2]
    cos = dot / jnp.maximum(jnp.sqrt(na2) * jnp.sqrt(nb2), 1e-8)
    return jnp.mean(1.0 - cos)


# speedup vs baseline: 1.1851x; 1.1851x over previous
import functools

import jax
import jax.numpy as jnp
from jax.experimental import pallas as pl
from jax.experimental.pallas import tpu as pltpu

_NH = 2


def _loss_kernel(en_ref, de_ref, out_ref, acc_ref, *, nh):
    b = pl.program_id(0)
    j = pl.program_id(1)

    @pl.when(j == 0)
    def _init():
        acc_ref[0] = 0.0
        acc_ref[1] = 0.0
        acc_ref[2] = 0.0

    en = en_ref[0]
    de = de_ref[0]

    acc_ref[0] += jnp.sum(en * de)
    acc_ref[1] += jnp.sum(en * en)
    acc_ref[2] += jnp.sum(de * de)

    @pl.when(j == nh - 1)
    def _finalize():
        dot, na2, nb2 = acc_ref[0], acc_ref[1], acc_ref[2]
        term = 1.0 - dot / jnp.maximum(jnp.sqrt(na2) * jnp.sqrt(nb2), 1e-8)

        @pl.when(b == 0)
        def _first():
            out_ref[0, 0] = term

        @pl.when(b > 0)
        def _rest():
            out_ref[0, 0] += term


def kernel(encoder_features, decoder_features, global_step):
    B, C, H, W = encoder_features.shape
    en = jnp.transpose(encoder_features, (0, 2, 3, 1))
    de = jnp.transpose(decoder_features, (0, 2, 3, 1))
    hb = H // _NH

    out = pl.pallas_call(
        functools.partial(_loss_kernel, nh=_NH),
        grid=(B, _NH),
        in_specs=[
            pl.BlockSpec((1, hb, W, C), lambda b, j: (b, j, 0, 0)),
            pl.BlockSpec((1, hb, W, C), lambda b, j: (b, j, 0, 0)),
        ],
        out_specs=pl.BlockSpec(
            (1, 1), lambda b, j: (0, 0), memory_space=pltpu.SMEM
        ),
        out_shape=jax.ShapeDtypeStruct((1, 1), jnp.float32),
        scratch_shapes=[pltpu.SMEM((3,), jnp.float32)],
    )(en, de)
    return (out[0, 0] / B).reshape(())


# speedup vs baseline: 1.5155x; 1.2787x over previous
import jax
import jax.numpy as jnp
from jax.experimental import pallas as pl


def _loss_kernel(en_ref, de_ref, out_ref):
    en = en_ref[0]
    de = de_ref[0]
    out_ref[0, 0] = jnp.sum(en * de, axis=(0, 1))
    out_ref[0, 1] = jnp.sum(en * en, axis=(0, 1))
    out_ref[0, 2] = jnp.sum(de * de, axis=(0, 1))


def kernel(encoder_features, decoder_features, global_step):
    B, C, H, W = encoder_features.shape
    en = jnp.transpose(encoder_features, (0, 2, 3, 1))
    de = jnp.transpose(decoder_features, (0, 2, 3, 1))

    out = pl.pallas_call(
        _loss_kernel,
        grid=(B,),
        in_specs=[
            pl.BlockSpec((1, H, W, C), lambda b: (b, 0, 0, 0)),
            pl.BlockSpec((1, H, W, C), lambda b: (b, 0, 0, 0)),
        ],
        out_specs=pl.BlockSpec((1, 3, C), lambda b: (b, 0, 0)),
        out_shape=jax.ShapeDtypeStruct((B, 3, C), jnp.float32),
    )(en, de)

    tot = out.sum(-1)
    dot, na2, nb2 = tot[:, 0], tot[:, 1], tot[:, 2]
    cos = dot / jnp.maximum(jnp.sqrt(na2) * jnp.sqrt(nb2), 1e-8)
    return jnp.mean(1.0 - cos)
